# baseline (speedup 1.0000x reference)
import jax
import jax.numpy as jnp
from jax.experimental import pallas as pl
from jax.experimental.pallas import tpu as pltpu

D = 768
TILE = 512

_INV_SQRT2 = 0.7071067811865476
_DNT = (((1,), (1,)), ((), ()))


def _gelu_exact(x):
    return 0.5 * x * (1.0 + jax.lax.erf(x * _INV_SQRT2))


def _mmt(x, w):
    return jax.lax.dot_general(x, w, _DNT, preferred_element_type=jnp.float32)


def _body(t_ref, o_ref, ms_ref,
          w1_ref, w2_ref, w3_ref, w4_ref,
          g1_ref, gb1_ref, b1p_ref, c2_ref,
          g2_ref, gb2_ref, b2p_ref, c4_ref,
          gf_ref, bf_ref,
          out_ref,
          w1s_ref, w2s_ref, w3s_ref, w4s_ref, vu1_ref, vu3_ref):
    bf = jnp.bfloat16
    i = pl.program_id(0)

    @pl.when(i == 0)
    def _prep():
        w1s_ref[...] = (w1_ref[...] * g1_ref[0, :]).astype(bf)
        w2s_ref[...] = w2_ref[...].astype(bf)
        w3s_ref[...] = (w3_ref[...] * g2_ref[0, :]).astype(bf)
        w4s_ref[...] = w4_ref[...].astype(bf)
        vu1_ref[...] = _mmt(gb1_ref[...], w1_ref[...]) + b1p_ref[...]
        vu3_ref[...] = _mmt(gb2_ref[...], w3_ref[...]) + b2p_ref[...]

    t = t_ref[...]
    o = o_ref[...]
    m = ms_ref[:, 0:1]
    osc = ms_ref[:, 1:2]

    st = jnp.sum(t, axis=1, keepdims=True)
    qt = jnp.sum(t * t, axis=1, keepdims=True)
    so = jnp.sum(o, axis=1, keepdims=True)
    qo = jnp.sum(o * o, axis=1, keepdims=True)
    mu = (st + osc * so) / (2 * D)
    var = (qt + osc * osc * qo) / (2 * D) - mu * mu
    inv = jax.lax.rsqrt(var + 1e-5)

    p = (_mmt(t.astype(bf), w1s_ref[:, :D])
         + osc * _mmt(o.astype(bf), w1s_ref[:, D:]))
    h = (inv.astype(bf) * (p.astype(bf) - mu.astype(bf) * vu1_ref[0:1, :].astype(bf))
         + vu1_ref[1:2, :].astype(bf))
    h = _gelu_exact(h)
    x1 = _mmt(h, w2s_ref[...]) + c2_ref[0, :] + t

    s1 = jnp.sum(x1, axis=1, keepdims=True)
    q1 = jnp.sum(x1 * x1, axis=1, keepdims=True)
    mu2 = s1 / D
    inv2 = jax.lax.rsqrt(q1 / D - mu2 * mu2 + 1e-5)
    p2 = _mmt(x1.astype(bf), w3s_ref[...])
    h2 = (inv2.astype(bf) * (p2.astype(bf) - mu2.astype(bf) * vu3_ref[0:1, :].astype(bf))
          + vu3_ref[1:2, :].astype(bf))
    h2 = _gelu_exact(h2)
    x2 = _mmt(h2, w4s_ref[...]) + c4_ref[0, :] + x1

    mu3 = jnp.mean(x2, axis=1, keepdims=True)
    var3 = jnp.mean(x2 * x2, axis=1, keepdims=True) - mu3 * mu3
    y = (x2 - mu3) * jax.lax.rsqrt(var3 + 1e-5) * gf_ref[0, :] + bf_ref[0, :]

    out_ref[...] = jnp.where(m > 0, y, t)


def kernel(text_embeds, object_embeds, image_token_mask, num_objects,
           mlp1_ln_g, mlp1_ln_b, mlp1_fc1_w, mlp1_fc1_b, mlp1_fc2_w, mlp1_fc2_b,
           mlp2_ln_g, mlp2_ln_b, mlp2_fc1_w, mlp2_fc1_b, mlp2_fc2_w, mlp2_fc2_b,
           ln_g, ln_b):
    B, S, _ = text_embeds.shape
    N = B * S
    nb = N // TILE
    bf = jnp.bfloat16
    f32 = jnp.float32

    t = text_embeds.reshape(N, D)
    o = object_embeds.reshape(N, D)
    ms = jnp.stack(
        [image_token_mask.reshape(N).astype(f32),
         jnp.repeat((num_objects > 0).astype(f32), S)], axis=1)

    gb1 = jnp.zeros((8, 2 * D), f32).at[0].set(mlp1_ln_g).at[1].set(mlp1_ln_b)
    b1p = jnp.zeros((8, D), f32).at[1].set(mlp1_fc1_b)
    gb2 = jnp.zeros((8, D), f32).at[0].set(mlp2_ln_g).at[1].set(mlp2_ln_b)
    b2p = jnp.zeros((8, D), f32).at[1].set(mlp2_fc1_b)

    full = lambda shape: pl.BlockSpec(shape, lambda i: (0,) * len(shape))
    out = pl.pallas_call(
        _body,
        grid=(nb,),
        in_specs=[
            pl.BlockSpec((TILE, D), lambda i: (i, 0)),
            pl.BlockSpec((TILE, D), lambda i: (i, 0)),
            pl.BlockSpec((TILE, 2), lambda i: (i, 0)),
            full((D, 2 * D)), full((D, D)), full((D, D)), full((D, D)),
            full((1, 2 * D)), full((8, 2 * D)), full((8, D)), full((1, D)),
            full((1, D)), full((8, D)), full((8, D)), full((1, D)),
            full((1, D)), full((1, D)),
        ],
        out_specs=pl.BlockSpec((TILE, D), lambda i: (i, 0)),
        out_shape=jax.ShapeDtypeStruct((N, D), jnp.float32),
        scratch_shapes=[
            pltpu.VMEM((D, 2 * D), bf), pltpu.VMEM((D, D), bf),
            pltpu.VMEM((D, D), bf), pltpu.VMEM((D, D), bf),
            pltpu.VMEM((8, D), f32), pltpu.VMEM((8, D), f32),
        ],
        compiler_params=pltpu.CompilerParams(
            dimension_semantics=("arbitrary",)),
    )(t, o, ms,
      mlp1_fc1_w, mlp1_fc2_w, mlp2_fc1_w, mlp2_fc2_w,
      mlp1_ln_g.reshape(1, 2 * D), gb1, b1p, mlp1_fc2_b.reshape(1, D),
      mlp2_ln_g.reshape(1, D), gb2, b2p, mlp2_fc2_b.reshape(1, D),
      ln_g.reshape(1, D), ln_b.reshape(1, D))
    return out.reshape(B, S, D)


# speedup vs baseline: 1.0250x; 1.0250x over previous
import jax
import jax.numpy as jnp
from jax.experimental import pallas as pl
from jax.experimental.pallas import tpu as pltpu

D = 768
TILE = 1024

_INV_SQRT2 = 0.7071067811865476
_DNT = (((1,), (1,)), ((), ()))


def _gelu_exact(x):
    return 0.5 * x * (1.0 + jax.lax.erf(x * _INV_SQRT2))


def _mmt(x, w):
    return jax.lax.dot_general(x, w, _DNT, preferred_element_type=jnp.float32)


def _body(t_ref, o_ref, ms_ref,
          w1_ref, w2_ref, w3_ref, w4_ref,
          g1_ref, gb1_ref, b1p_ref, c2_ref,
          g2_ref, gb2_ref, b2p_ref, c4_ref,
          gf_ref, bf_ref,
          out_ref,
          w1s_ref, w2s_ref, w3s_ref, w4s_ref, vu1_ref, vu3_ref):
    bf = jnp.bfloat16
    i = pl.program_id(0)

    @pl.when(i == 0)
    def _prep():
        w1s_ref[...] = (w1_ref[...] * g1_ref[0, :]).astype(bf)
        w2s_ref[...] = w2_ref[...].astype(bf)
        w3s_ref[...] = (w3_ref[...] * g2_ref[0, :]).astype(bf)
        w4s_ref[...] = w4_ref[...].astype(bf)
        vu1_ref[...] = _mmt(gb1_ref[...], w1_ref[...]) + b1p_ref[...]
        vu3_ref[...] = _mmt(gb2_ref[...], w3_ref[...]) + b2p_ref[...]

    t = t_ref[...]
    o = o_ref[...]
    m = ms_ref[:, 0:1]
    osc = ms_ref[:, 1:2]

    st = jnp.sum(t, axis=1, keepdims=True)
    qt = jnp.sum(t * t, axis=1, keepdims=True)
    so = jnp.sum(o, axis=1, keepdims=True)
    qo = jnp.sum(o * o, axis=1, keepdims=True)
    mu = (st + osc * so) / (2 * D)
    var = (qt + osc * osc * qo) / (2 * D) - mu * mu
    inv = jax.lax.rsqrt(var + 1e-5)

    p = (_mmt(t.astype(bf), w1s_ref[:, :D])
         + osc * _mmt(o.astype(bf), w1s_ref[:, D:]))
    h = (inv.astype(bf) * (p.astype(bf) - mu.astype(bf) * vu1_ref[0:1, :].astype(bf))
         + vu1_ref[1:2, :].astype(bf))
    h = _gelu_exact(h)
    x1 = _mmt(h, w2s_ref[...]) + c2_ref[0, :] + t

    s1 = jnp.sum(x1, axis=1, keepdims=True)
    q1 = jnp.sum(x1 * x1, axis=1, keepdims=True)
    mu2 = s1 / D
    inv2 = jax.lax.rsqrt(q1 / D - mu2 * mu2 + 1e-5)
    p2 = _mmt(x1.astype(bf), w3s_ref[...])
    h2 = (inv2.astype(bf) * (p2.astype(bf) - mu2.astype(bf) * vu3_ref[0:1, :].astype(bf))
          + vu3_ref[1:2, :].astype(bf))
    h2 = _gelu_exact(h2)
    x2 = _mmt(h2, w4s_ref[...]) + c4_ref[0, :] + x1

    mu3 = jnp.mean(x2, axis=1, keepdims=True)
    var3 = jnp.mean(x2 * x2, axis=1, keepdims=True) - mu3 * mu3
    y = (x2 - mu3) * jax.lax.rsqrt(var3 + 1e-5) * gf_ref[0, :] + bf_ref[0, :]

    out_ref[...] = jnp.where(m > 0, y, t)


def kernel(text_embeds, object_embeds, image_token_mask, num_objects,
           mlp1_ln_g, mlp1_ln_b, mlp1_fc1_w, mlp1_fc1_b, mlp1_fc2_w, mlp1_fc2_b,
           mlp2_ln_g, mlp2_ln_b, mlp2_fc1_w, mlp2_fc1_b, mlp2_fc2_w, mlp2_fc2_b,
           ln_g, ln_b):
    B, S, _ = text_embeds.shape
    N = B * S
    nb = N // TILE
    bf = jnp.bfloat16
    f32 = jnp.float32

    t = text_embeds.reshape(N, D)
    o = object_embeds.reshape(N, D)
    ms = jnp.stack(
        [image_token_mask.reshape(N).astype(f32),
         jnp.repeat((num_objects > 0).astype(f32), S)], axis=1)

    gb1 = jnp.zeros((8, 2 * D), f32).at[0].set(mlp1_ln_g).at[1].set(mlp1_ln_b)
    b1p = jnp.zeros((8, D), f32).at[1].set(mlp1_fc1_b)
    gb2 = jnp.zeros((8, D), f32).at[0].set(mlp2_ln_g).at[1].set(mlp2_ln_b)
    b2p = jnp.zeros((8, D), f32).at[1].set(mlp2_fc1_b)

    full = lambda shape: pl.BlockSpec(shape, lambda i: (0,) * len(shape))
    out = pl.pallas_call(
        _body,
        grid=(nb,),
        in_specs=[
            pl.BlockSpec((TILE, D), lambda i: (i, 0)),
            pl.BlockSpec((TILE, D), lambda i: (i, 0)),
            pl.BlockSpec((TILE, 2), lambda i: (i, 0)),
            full((D, 2 * D)), full((D, D)), full((D, D)), full((D, D)),
            full((1, 2 * D)), full((8, 2 * D)), full((8, D)), full((1, D)),
            full((1, D)), full((8, D)), full((8, D)), full((1, D)),
            full((1, D)), full((1, D)),
        ],
        out_specs=pl.BlockSpec((TILE, D), lambda i: (i, 0)),
        out_shape=jax.ShapeDtypeStruct((N, D), jnp.float32),
        scratch_shapes=[
            pltpu.VMEM((D, 2 * D), bf), pltpu.VMEM((D, D), bf),
            pltpu.VMEM((D, D), bf), pltpu.VMEM((D, D), bf),
            pltpu.VMEM((8, D), f32), pltpu.VMEM((8, D), f32),
        ],
        compiler_params=pltpu.CompilerParams(
            dimension_semantics=("arbitrary",)),
    )(t, o, ms,
      mlp1_fc1_w, mlp1_fc2_w, mlp2_fc1_w, mlp2_fc2_w,
      mlp1_ln_g.reshape(1, 2 * D), gb1, b1p, mlp1_fc2_b.reshape(1, D),
      mlp2_ln_g.reshape(1, D), gb2, b2p, mlp2_fc2_b.reshape(1, D),
      ln_g.reshape(1, D), ln_b.reshape(1, D))
    return out.reshape(B, S, D)
